# baseline (speedup 1.0000x reference)
import functools

import jax
import jax.numpy as jnp
from jax import lax
from jax.experimental import pallas as pl
from jax.experimental.pallas import tpu as pltpu
from jax.experimental.pallas import tpu_sc as plsc

N = 128 * 128 * 128
NC = 2
NS = 16
NW = NC * NS
LANES = 16
UNROLL = 4

SLABS = 128
SC_SLABS = 32
TC_SLABS = SLABS - SC_SLABS
TC_BLOCK = 32

SC_N = SC_SLABS * 16384
PER_W = SC_N // NW if SC_N else 0
CHUNK = min(PER_W, 8192) if SC_N else 0
NCHUNK = (PER_W // CHUNK) if SC_N else 0

_mesh = plsc.VectorSubcoreMesh(core_axis_name="c", subcore_axis_name="s")


def _sc_body(pred_hbm, targ_hbm, out_hbm,
             p0, g0, m0, p1, g1, m1, acc_v, sem0, sem1):
    wid = lax.axis_index("s") * NC + lax.axis_index("c")
    base = wid * PER_W
    bufs = ((p0, g0, m0), (p1, g1, m1))
    sems = (sem0, sem1)

    def issue(j, slot):
        off = base + j * CHUNK
        pv, gv, mv = bufs[slot]
        return (
            pltpu.async_copy(pred_hbm.at[pl.ds(off, CHUNK)], pv, sems[slot]),
            pltpu.async_copy(targ_hbm.at[pl.ds(off, CHUNK)], gv, sems[slot]),
            pltpu.async_copy(targ_hbm.at[pl.ds(N + off, CHUNK)], mv, sems[slot]),
        )

    accs = [jnp.zeros((LANES,), jnp.float32) for _ in range(UNROLL)]
    cnts = [jnp.zeros((LANES,), jnp.float32) for _ in range(UNROLL)]

    pending = [None, None]
    pending[0] = issue(0, 0)
    for j in range(NCHUNK):
        slot = j & 1
        if j + 1 < NCHUNK:
            pending[(j + 1) & 1] = issue(j + 1, (j + 1) & 1)
        for d in pending[slot]:
            d.wait()
        pv, gv, mv = bufs[slot]

        def body(i, carry):
            a = list(carry[:UNROLL])
            c = list(carry[UNROLL:])
            for u in range(UNROLL):
                s = i * (LANES * UNROLL) + u * LANES
                p = pv[pl.ds(s, LANES)]
                g = gv[pl.ds(s, LANES)]
                m = mv[pl.ds(s, LANES)]
                sel = m > 0
                a[u] = a[u] + jnp.where(sel, jnp.abs(p - g), 0.0)
                c[u] = c[u] + jnp.where(sel, 1.0, 0.0)
            return tuple(a) + tuple(c)

        out = lax.fori_loop(0, CHUNK // (LANES * UNROLL), body,
                            tuple(accs) + tuple(cnts))
        accs = list(out[:UNROLL])
        cnts = list(out[UNROLL:])

    acc = accs[0] + accs[1] + accs[2] + accs[3]
    cnt = cnts[0] + cnts[1] + cnts[2] + cnts[3]
    acc_v[pl.ds(0, LANES)] = acc
    acc_v[pl.ds(LANES, LANES)] = cnt
    pltpu.sync_copy(acc_v, out_hbm.at[wid])


if SC_N:
    _sc_partials = functools.partial(
        pl.kernel,
        out_type=jax.ShapeDtypeStruct((NW, 2 * LANES), jnp.float32),
        mesh=_mesh,
        scratch_types=[
            pltpu.VMEM((CHUNK,), jnp.float32),
            pltpu.VMEM((CHUNK,), jnp.float32),
            pltpu.VMEM((CHUNK,), jnp.float32),
            pltpu.VMEM((CHUNK,), jnp.float32),
            pltpu.VMEM((CHUNK,), jnp.float32),
            pltpu.VMEM((CHUNK,), jnp.float32),
            pltpu.VMEM((2 * LANES,), jnp.float32),
            pltpu.SemaphoreType.DMA,
            pltpu.SemaphoreType.DMA,
        ],
    )(_sc_body)


def _tc_body(p_ref, t_ref, out_ref, acc_ref, cnt_ref):
    i = pl.program_id(0)
    n = pl.num_programs(0)

    sel = t_ref[1, 0] > 0
    d = jnp.sum(jnp.where(sel, jnp.abs(p_ref[0, 0] - t_ref[0, 0]), 0.0), axis=0)
    one = jnp.sum(jnp.where(sel, 1.0, 0.0), axis=0)

    @pl.when(i == 0)
    def _init():
        acc_ref[...] = d
        cnt_ref[...] = one

    @pl.when(i > 0)
    def _accum():
        acc_ref[...] += d
        cnt_ref[...] += one

    @pl.when(i == n - 1)
    def _final():
        total = jnp.sum(acc_ref[...])
        count = jnp.sum(cnt_ref[...])
        if SC_N:
            out_ref[0] = total
            out_ref[1] = count
        else:
            out_ref[0] = total / jnp.maximum(count, 1.0)
            out_ref[1] = count


_tc_sums = pl.pallas_call(
    _tc_body,
    grid=(TC_SLABS // TC_BLOCK,),
    in_specs=[
        pl.BlockSpec((1, 1, TC_BLOCK, 128, 128),
                     lambda i: (0, 0, i + SC_SLABS // TC_BLOCK, 0, 0)),
        pl.BlockSpec((2, 1, TC_BLOCK, 128, 128),
                     lambda i: (0, 0, i + SC_SLABS // TC_BLOCK, 0, 0)),
    ],
    out_specs=pl.BlockSpec(memory_space=pltpu.SMEM),
    out_shape=jax.ShapeDtypeStruct((2,), jnp.float32),
    scratch_shapes=[
        pltpu.VMEM((128, 128), jnp.float32),
        pltpu.VMEM((128, 128), jnp.float32),
    ],
)


@jax.jit
def kernel(predictions, targets):
    total = jnp.float32(0)
    count = jnp.float32(0)

    if SC_N:
        pred_flat = predictions.reshape(-1)
        targ_flat = targets.reshape(-1)
        partials = _sc_partials(pred_flat, targ_flat)
        total += jnp.sum(partials[:, :LANES])
        count += jnp.sum(partials[:, LANES:])

    if TC_SLABS:
        tc = _tc_sums(predictions, targets)
        if not SC_N:
            return tc[0]
        total += tc[0]
        count += tc[1]

    return total / jnp.maximum(count, 1.0)


# speedup vs baseline: 2.9646x; 2.9646x over previous
import jax
import jax.numpy as jnp
from jax.experimental import pallas as pl
from jax.experimental.pallas import tpu as pltpu

SLABS = 128
TC_BLOCK = 32


def _loss_body(p_ref, t_ref, out_ref, acc_ref, cnt_ref):
    i = pl.program_id(0)
    n = pl.num_programs(0)

    sel = t_ref[1, 0] > 0
    d = jnp.sum(jnp.where(sel, jnp.abs(p_ref[0, 0] - t_ref[0, 0]), 0.0), axis=0)
    one = jnp.sum(jnp.where(sel, 1.0, 0.0), axis=0)

    @pl.when(i == 0)
    def _init():
        acc_ref[...] = d
        cnt_ref[...] = one

    @pl.when(i > 0)
    def _accum():
        acc_ref[...] += d
        cnt_ref[...] += one

    @pl.when(i == n - 1)
    def _final():
        total = jnp.sum(acc_ref[...])
        count = jnp.sum(cnt_ref[...])
        out_ref[0] = total / jnp.maximum(count, 1.0)


_loss = pl.pallas_call(
    _loss_body,
    grid=(SLABS // TC_BLOCK,),
    in_specs=[
        pl.BlockSpec((1, 1, TC_BLOCK, 128, 128), lambda i: (0, 0, i, 0, 0)),
        pl.BlockSpec((2, 1, TC_BLOCK, 128, 128), lambda i: (0, 0, i, 0, 0)),
    ],
    out_specs=pl.BlockSpec(memory_space=pltpu.SMEM),
    out_shape=jax.ShapeDtypeStruct((1,), jnp.float32),
    scratch_shapes=[
        pltpu.VMEM((128, 128), jnp.float32),
        pltpu.VMEM((128, 128), jnp.float32),
    ],
)


@jax.jit
def kernel(predictions, targets):
    return _loss(predictions, targets)[0]


# speedup vs baseline: 2.9653x; 1.0002x over previous
import jax
import jax.numpy as jnp
from jax.experimental import pallas as pl
from jax.experimental.pallas import tpu as pltpu

CHUNKS = (8, 24, 32, 32, 32)
STARTS = (0, 8, 32, 64, 96)
MAXW = 32


def _loss_body(p_hbm, t_hbm, out_ref,
               pb0, gb0, mb0, pb1, gb1, mb1, acc_ref, cnt_ref, sem0, sem1):
    bufs = ((pb0, gb0, mb0), (pb1, gb1, mb1))
    sems = (sem0, sem1)

    def issue(ci, slot):
        s, w = STARTS[ci], CHUNKS[ci]
        pb, gb, mb = bufs[slot]
        cps = (
            pltpu.make_async_copy(p_hbm.at[pl.ds(s, w), :, :],
                                  pb.at[pl.ds(0, w), :, :], sems[slot]),
            pltpu.make_async_copy(t_hbm.at[pl.ds(s, w), :, :],
                                  gb.at[pl.ds(0, w), :, :], sems[slot]),
            pltpu.make_async_copy(t_hbm.at[pl.ds(128 + s, w), :, :],
                                  mb.at[pl.ds(0, w), :, :], sems[slot]),
        )
        for c in cps:
            c.start()
        return cps

    pending = [None, None]
    pending[0] = issue(0, 0)
    pending[1] = issue(1, 1)

    for ci in range(len(CHUNKS)):
        slot = ci & 1
        w = CHUNKS[ci]
        for c in pending[slot]:
            c.wait()
        pb, gb, mb = bufs[slot]
        sel = mb[:w] > 0
        d = jnp.sum(jnp.where(sel, jnp.abs(pb[:w] - gb[:w]), 0.0), axis=0)
        one = jnp.sum(jnp.where(sel, 1.0, 0.0), axis=0)
        if ci == 0:
            acc_ref[...] = d
            cnt_ref[...] = one
        else:
            acc_ref[...] += d
            cnt_ref[...] += one
        if ci + 2 < len(CHUNKS):
            pending[slot] = issue(ci + 2, slot)

    total = jnp.sum(acc_ref[...])
    count = jnp.sum(cnt_ref[...])
    out_ref[0] = total / jnp.maximum(count, 1.0)


_loss = pl.pallas_call(
    _loss_body,
    in_specs=[
        pl.BlockSpec(memory_space=pl.ANY),
        pl.BlockSpec(memory_space=pl.ANY),
    ],
    out_specs=pl.BlockSpec(memory_space=pltpu.SMEM),
    out_shape=jax.ShapeDtypeStruct((1,), jnp.float32),
    scratch_shapes=[
        pltpu.VMEM((MAXW, 128, 128), jnp.float32),
        pltpu.VMEM((MAXW, 128, 128), jnp.float32),
        pltpu.VMEM((MAXW, 128, 128), jnp.float32),
        pltpu.VMEM((MAXW, 128, 128), jnp.float32),
        pltpu.VMEM((MAXW, 128, 128), jnp.float32),
        pltpu.VMEM((MAXW, 128, 128), jnp.float32),
        pltpu.VMEM((128, 128), jnp.float32),
        pltpu.VMEM((128, 128), jnp.float32),
        pltpu.SemaphoreType.DMA,
        pltpu.SemaphoreType.DMA,
    ],
)


@jax.jit
def kernel(predictions, targets):
    pred3 = predictions.reshape(256, 128, 128)
    targ3 = targets.reshape(256, 128, 128)
    return _loss(pred3, targ3)[0]


# speedup vs baseline: 3.1972x; 1.0782x over previous
import jax
import jax.numpy as jnp
from jax.experimental import pallas as pl
from jax.experimental.pallas import tpu as pltpu

W = 16
NCHUNK = 128 // W
NSLOT = 4
AHEAD = 3


def _loss_body(p_hbm, t_hbm, out_ref, *rest):
    bufs = tuple((rest[3 * k], rest[3 * k + 1], rest[3 * k + 2])
                 for k in range(NSLOT))
    acc_ref = rest[3 * NSLOT]
    cnt_ref = rest[3 * NSLOT + 1]
    sems = rest[3 * NSLOT + 2:]

    def issue(ci):
        slot = ci % NSLOT
        s = ci * W
        pb, gb, mb = bufs[slot]
        cps = (
            pltpu.make_async_copy(p_hbm.at[pl.ds(s, W), :, :], pb, sems[slot]),
            pltpu.make_async_copy(t_hbm.at[pl.ds(s, W), :, :], gb, sems[slot]),
            pltpu.make_async_copy(t_hbm.at[pl.ds(128 + s, W), :, :], mb, sems[slot]),
        )
        for c in cps:
            c.start()
        return cps

    pending = [None] * NSLOT
    for ci in range(AHEAD):
        pending[ci % NSLOT] = issue(ci)

    for ci in range(NCHUNK):
        slot = ci % NSLOT
        for c in pending[slot]:
            c.wait()
        pb, gb, mb = bufs[slot]
        sel = mb[...] > 0
        d = jnp.sum(jnp.where(sel, jnp.abs(pb[...] - gb[...]), 0.0), axis=0)
        one = jnp.sum(jnp.where(sel, 1.0, 0.0), axis=0)
        if ci == 0:
            acc_ref[...] = d
            cnt_ref[...] = one
        else:
            acc_ref[...] += d
            cnt_ref[...] += one
        if ci + AHEAD < NCHUNK:
            pending[(ci + AHEAD) % NSLOT] = issue(ci + AHEAD)

    total = jnp.sum(acc_ref[...])
    count = jnp.sum(cnt_ref[...])
    out_ref[0] = total / jnp.maximum(count, 1.0)


_loss = pl.pallas_call(
    _loss_body,
    in_specs=[
        pl.BlockSpec(memory_space=pl.ANY),
        pl.BlockSpec(memory_space=pl.ANY),
    ],
    out_specs=pl.BlockSpec(memory_space=pltpu.SMEM),
    out_shape=jax.ShapeDtypeStruct((1,), jnp.float32),
    scratch_shapes=(
        [pltpu.VMEM((W, 128, 128), jnp.float32) for _ in range(3 * NSLOT)]
        + [pltpu.VMEM((128, 128), jnp.float32),
           pltpu.VMEM((128, 128), jnp.float32)]
        + [pltpu.SemaphoreType.DMA for _ in range(NSLOT)]
    ),
)


@jax.jit
def kernel(predictions, targets):
    pred3 = predictions.reshape(256, 128, 128)
    targ3 = targets.reshape(256, 128, 128)
    return _loss(pred3, targ3)[0]


# speedup vs baseline: 3.2922x; 1.0297x over previous
import jax
import jax.numpy as jnp
from jax.experimental import pallas as pl
from jax.experimental.pallas import tpu as pltpu

W = 16
NCHUNK = 128 // W
NSLOT = 6
AHEAD = 5


def _loss_body(p_hbm, t_hbm, out_ref, *rest):
    bufs = tuple((rest[3 * k], rest[3 * k + 1], rest[3 * k + 2])
                 for k in range(NSLOT))
    acc_ref = rest[3 * NSLOT]
    cnt_ref = rest[3 * NSLOT + 1]
    sems = rest[3 * NSLOT + 2:]

    def issue(ci):
        slot = ci % NSLOT
        s = ci * W
        pb, gb, mb = bufs[slot]
        cps = (
            pltpu.make_async_copy(p_hbm.at[pl.ds(s, W), :, :], pb, sems[slot]),
            pltpu.make_async_copy(t_hbm.at[pl.ds(s, W), :, :], gb, sems[slot]),
            pltpu.make_async_copy(t_hbm.at[pl.ds(128 + s, W), :, :], mb, sems[slot]),
        )
        for c in cps:
            c.start()
        return cps

    pending = [None] * NSLOT
    for ci in range(AHEAD):
        pending[ci % NSLOT] = issue(ci)

    for ci in range(NCHUNK):
        slot = ci % NSLOT
        for c in pending[slot]:
            c.wait()
        pb, gb, mb = bufs[slot]
        sel = mb[...] > 0
        d = jnp.sum(jnp.where(sel, jnp.abs(pb[...] - gb[...]), 0.0), axis=0)
        one = jnp.sum(jnp.where(sel, 1.0, 0.0), axis=0)
        if ci == 0:
            acc_ref[...] = d
            cnt_ref[...] = one
        else:
            acc_ref[...] += d
            cnt_ref[...] += one
        if ci + AHEAD < NCHUNK:
            pending[(ci + AHEAD) % NSLOT] = issue(ci + AHEAD)

    total = jnp.sum(acc_ref[...])
    count = jnp.sum(cnt_ref[...])
    out_ref[0] = total / jnp.maximum(count, 1.0)


_loss = pl.pallas_call(
    _loss_body,
    in_specs=[
        pl.BlockSpec(memory_space=pl.ANY),
        pl.BlockSpec(memory_space=pl.ANY),
    ],
    out_specs=pl.BlockSpec(memory_space=pltpu.SMEM),
    out_shape=jax.ShapeDtypeStruct((1,), jnp.float32),
    scratch_shapes=(
        [pltpu.VMEM((W, 128, 128), jnp.float32) for _ in range(3 * NSLOT)]
        + [pltpu.VMEM((128, 128), jnp.float32),
           pltpu.VMEM((128, 128), jnp.float32)]
        + [pltpu.SemaphoreType.DMA for _ in range(NSLOT)]
    ),
)


@jax.jit
def kernel(predictions, targets):
    pred3 = predictions.reshape(256, 128, 128)
    targ3 = targets.reshape(256, 128, 128)
    return _loss(pred3, targ3)[0]
